# baseline (speedup 1.0000x reference)
import jax
import jax.numpy as jnp
from jax.experimental import pallas as pl

B, N = 128, 100000
ROWS = 8
STEPS = B // ROWS


def _body(lg_ref, mk_ref, gm_ref, act_ref, lp_ref):
    lg = lg_ref[...]
    mk = mk_ref[...]
    gm = gm_ref[...]

    neg = jnp.float32(-1e9)
    ml = jnp.where(mk, lg, neg)
    m = jnp.max(ml, axis=1, keepdims=True)
    e = jnp.exp(ml - m)
    em = jnp.where(mk, e, 0.0)
    Z = jnp.sum(e, axis=1, keepdims=True)
    Ssum = jnp.sum(em, axis=1, keepdims=True)
    S = Ssum / Z
    invC = 1.0 / (Z * (S + jnp.float32(1e-8)))
    v = jnp.log(em * invC + jnp.float32(1e-9)) + gm

    a = jnp.argmax(v, axis=1).astype(jnp.int32)
    vmax = jnp.max(v, axis=1)
    iota = jax.lax.broadcasted_iota(jnp.int32, v.shape, 1)
    g_at = jnp.sum(jnp.where(iota == a[:, None], gm, 0.0), axis=1)
    act_ref[0, 0, :] = a
    lp_ref[0, 0, :] = vmax - g_at


def kernel(logits, mask, gumbel):
    grid = (STEPS,)
    acts, lps = pl.pallas_call(
        _body,
        grid=grid,
        in_specs=[
            pl.BlockSpec((ROWS, N), lambda i: (i, 0)),
            pl.BlockSpec((ROWS, N), lambda i: (i, 0)),
            pl.BlockSpec((ROWS, N), lambda i: (i, 0)),
        ],
        out_specs=[
            pl.BlockSpec((1, 1, ROWS), lambda i: (i, 0, 0)),
            pl.BlockSpec((1, 1, ROWS), lambda i: (i, 0, 0)),
        ],
        out_shape=[
            jax.ShapeDtypeStruct((STEPS, 1, ROWS), jnp.int32),
            jax.ShapeDtypeStruct((STEPS, 1, ROWS), jnp.float32),
        ],
    )(logits, mask, gumbel)
    return acts.reshape(B), lps.reshape(B)
